# Initial kernel scaffold; baseline (speedup 1.0000x reference)
import jax
import jax.numpy as jnp
from jax.experimental import pallas as pl


def kernel(hidden_states, topk_weights, topk_ids, wi_0, wi_1, wo):
    raise NotImplementedError("write your pallas kernel here")


# speedup vs baseline: 5.7231x; 5.7231x over previous
import functools

import jax
import jax.numpy as jnp
from jax import lax
from jax.experimental import pallas as pl
from jax.experimental.pallas import tpu as pltpu
from jax.experimental.pallas import tpu_sc as plsc

E = 16
K = 2
D = 1024
F = 2048
T = 2048
M = T * K
TM = 128
NT = M // TM + E
MP = NT * TM
IDS_R = M // 128
IDS_C = 128

NC = 2
NS = 16
NW = NC * NS


def _routing_body(ids_ref, pos_ref, eot_ref):
    ids = ids_ref[...]
    li = lax.broadcasted_iota(jnp.int32, (IDS_C, IDS_C), 0)
    lj = lax.broadcasted_iota(jnp.int32, (IDS_C, IDS_C), 1)
    lmat = (li <= lj).astype(jnp.float32)
    ri = lax.broadcasted_iota(jnp.int32, (IDS_R, IDS_R), 0)
    rj = lax.broadcasted_iota(jnp.int32, (IDS_R, IDS_R), 1)
    tstrict = (rj < ri).astype(jnp.float32)
    pos = jnp.zeros((IDS_R, IDS_C), jnp.int32)
    off = jnp.int32(0)
    ends = []
    for e in range(E):
        m = ids == e
        mf = m.astype(jnp.float32)
        s = jnp.dot(mf, lmat, preferred_element_type=jnp.float32)
        row_tot = s[:, IDS_C - 1:IDS_C]
        excl = jnp.dot(tstrict, row_tot, preferred_element_type=jnp.float32)
        rank = (s - mf + excl).astype(jnp.int32)
        pos = jnp.where(m, off + rank, pos)
        tot = jnp.sum(mf).astype(jnp.int32)
        off = off + ((tot + TM - 1) // TM) * TM
        ends.append(off)
    pos_ref[...] = pos
    tstart = lax.broadcasted_iota(jnp.int32, (1, 128), 1) * TM
    eot = jnp.zeros((1, 128), jnp.int32)
    for e in range(E):
        eot = eot + (tstart >= ends[e]).astype(jnp.int32)
    eot_ref[...] = jnp.minimum(eot, E - 1)


_routing = pl.pallas_call(
    _routing_body,
    out_shape=(
        jax.ShapeDtypeStruct((IDS_R, IDS_C), jnp.int32),
        jax.ShapeDtypeStruct((1, 128), jnp.int32),
    ),
)


def _gmm_body(eot_ref, x_ref, w0_ref, w1_ref, wo_ref, o_ref):
    x = x_ref[...]
    a = jnp.dot(x, w0_ref[0], preferred_element_type=jnp.float32)
    b = jnp.dot(x, w1_ref[0], preferred_element_type=jnp.float32)
    h = a * jax.nn.sigmoid(a) * b
    o_ref[...] = jnp.dot(h, wo_ref[0], preferred_element_type=jnp.float32)


_gmm = pl.pallas_call(
    _gmm_body,
    grid_spec=pltpu.PrefetchScalarGridSpec(
        num_scalar_prefetch=1,
        grid=(NT,),
        in_specs=[
            pl.BlockSpec((TM, D), lambda t, eot: (t, 0)),
            pl.BlockSpec((1, D, F), lambda t, eot: (eot[t], 0, 0)),
            pl.BlockSpec((1, D, F), lambda t, eot: (eot[t], 0, 0)),
            pl.BlockSpec((1, F, D), lambda t, eot: (eot[t], 0, 0)),
        ],
        out_specs=pl.BlockSpec((TM, D), lambda t, eot: (t, 0)),
    ),
    out_shape=jax.ShapeDtypeStruct((MP, D), jnp.float32),
)


_TT = 256


def _combine_body(u_ref, tw_ref, o_ref):
    u = u_ref[...]
    w0c = tw_ref[:, 0:1]
    w1c = tw_ref[:, 1:2]
    o_ref[...] = u[:, :D] * w0c + u[:, D:] * w1c


_combine = pl.pallas_call(
    _combine_body,
    grid=(T // _TT,),
    in_specs=[
        pl.BlockSpec((_TT, K * D), lambda t: (t, 0)),
        pl.BlockSpec((_TT, K), lambda t: (t, 0)),
    ],
    out_specs=pl.BlockSpec((_TT, D), lambda t: (t, 0)),
    out_shape=jax.ShapeDtypeStruct((T, D), jnp.float32),
)


_ROWS_W = T // NW
_GROWS_W = M // NW
_GCHUNK = 64


@functools.cache
def _sc_kernels():
    mesh = plsc.VectorSubcoreMesh(
        core_axis_name="c", subcore_axis_name="s",
        num_cores=NC, num_subcores=NS)

    @functools.partial(
        pl.kernel,
        out_type=jax.ShapeDtypeStruct((MP, D), jnp.float32),
        mesh=mesh,
        scratch_types=[
            pltpu.VMEM((_ROWS_W, D), jnp.float32),
            pltpu.VMEM((_ROWS_W,), jnp.int32),
            pltpu.VMEM((_ROWS_W,), jnp.int32),
            pltpu.SemaphoreType.DMA,
        ],
    )
    def sc_permute(hid_hbm, pe_hbm, po_hbm, xp_hbm, rows_v, i0_v, i1_v, sem):
        wid = lax.axis_index("s") * NC + lax.axis_index("c")
        base = wid * _ROWS_W
        pltpu.sync_copy(hid_hbm.at[pl.ds(base, _ROWS_W)], rows_v)
        pltpu.sync_copy(pe_hbm.at[pl.ds(base, _ROWS_W)], i0_v)
        pltpu.sync_copy(po_hbm.at[pl.ds(base, _ROWS_W)], i1_v)
        pltpu.async_copy(rows_v, xp_hbm.at[i0_v], sem).wait()
        pltpu.async_copy(rows_v, xp_hbm.at[i1_v], sem).wait()

    @functools.partial(
        pl.kernel,
        out_type=jax.ShapeDtypeStruct((M, D), jnp.float32),
        mesh=mesh,
        scratch_types=[
            pltpu.VMEM((_GCHUNK, D), jnp.float32),
            pltpu.VMEM((_GCHUNK,), jnp.int32),
            pltpu.SemaphoreType.DMA,
        ],
    )
    def sc_gather(out2_hbm, posf_hbm, uns_hbm, rows_v, idx_v, sem):
        wid = lax.axis_index("s") * NC + lax.axis_index("c")
        for c in range(_GROWS_W // _GCHUNK):
            base = wid * _GROWS_W + c * _GCHUNK
            pltpu.sync_copy(posf_hbm.at[pl.ds(base, _GCHUNK)], idx_v)
            pltpu.async_copy(out2_hbm.at[idx_v], rows_v, sem).wait()
            pltpu.sync_copy(rows_v, uns_hbm.at[pl.ds(base, _GCHUNK)])

    return sc_permute, sc_gather


def kernel(hidden_states, topk_weights, topk_ids, wi_0, wi_1, wo):
    ids2 = topk_ids.reshape(IDS_R, IDS_C)
    pos, eot = _routing(ids2)
    pos_flat = pos.reshape(M)
    pos_even = pos_flat[0::2]
    pos_odd = pos_flat[1::2]
    eot_flat = eot.reshape(128)[:NT]
    sc_permute, sc_gather = _sc_kernels()
    x_pad = sc_permute(hidden_states, pos_even, pos_odd)
    out2 = _gmm(eot_flat, x_pad, wi_0, wi_1, wo)
    unsorted = sc_gather(out2, pos_flat)
    u2 = unsorted.reshape(T, K * D)
    return _combine(u2, topk_weights)
